# baseline (speedup 1.0000x reference)
import functools

import jax
import jax.numpy as jnp
from jax import lax
from jax.experimental import pallas as pl
from jax.experimental.pallas import tpu as pltpu
from jax.experimental.pallas import tpu_sc as plsc

N_NODES = 10000
D = 128
DH = D // 2
NC = 2
NS = 16
L = 16
CHUNK = 128
NBUF = 5
GAHEAD = 2
PBITS = 14
PMASK = (1 << PBITS) - 1

ROWS_PER_TILE = N_NODES // NS
ZROWS = 125


def _sc_body(n_chunks, pk_hbm, w_hbm, x_hbm, out_hbm,
             pk, wv, idxc, idxr, gb, shared, gsem, ssem):
    c = lax.axis_index("c")
    s = lax.axis_index("s")

    pltpu.sync_copy(pk_hbm.at[s], pk)
    pltpu.sync_copy(w_hbm.at[s], wv)

    zeros = jnp.zeros((L,), jnp.float32)

    def zrow(i, _):
        for v in range(DH // L):
            gb[0, i, pl.ds(v * L, L)] = zeros
        return 0
    lax.fori_loop(0, CHUNK, zrow, 0)
    for k in range(ROWS_PER_TILE // ZROWS):
        pltpu.sync_copy(gb.at[0, pl.ds(0, ZROWS), :],
                        shared.at[pl.ds(s * ROWS_PER_TILE + k * ZROWS, ZROWS), :])
    plsc.subcore_barrier()

    col_off = c * N_NODES

    def unpack(k, slot):
        for v in range(CHUNK // L):
            p = pk[pl.ds(k * CHUNK + v * L, L)]
            idxr[slot, pl.ds(v * L, L)] = lax.shift_right_logical(p, PBITS)
            idxc[slot, pl.ds(v * L, L)] = (p & PMASK) + col_off

    def gather_desc(j, b):
        return pltpu.make_async_copy(x_hbm.at[idxc.at[b]], gb.at[b], gsem)

    def scatter_desc(j, b):
        return pltpu.make_async_copy(gb.at[b], shared.at[idxr.at[b]], ssem)

    for b in range(GAHEAD):
        unpack(b, b)
        gather_desc(b, b).start()

    def outer(jo, _):
        for b in range(NBUF):
            j = jo * NBUF + b
            gather_desc(j, b).wait()

            @pl.when(j >= NBUF - GAHEAD)
            def _():
                scatter_desc(j - (NBUF - GAHEAD), (b + GAHEAD) % NBUF).wait()

            @pl.when(j + GAHEAD < n_chunks)
            def _():
                unpack(j + GAHEAD, (b + GAHEAD) % NBUF)
                gather_desc(j + GAHEAD, (b + GAHEAD) % NBUF).start()

            jbase = j * CHUNK

            def escale(e, _):
                w16 = plsc.load_gather(wv, [jnp.broadcast_to(jbase + e, (L,))])
                for v in range(DH // L):
                    gb[b, e, pl.ds(v * L, L)] = gb[b, e, pl.ds(v * L, L)] * w16
                return 0
            lax.fori_loop(0, CHUNK, escale, 0, unroll=2)
            pltpu.async_copy(gb.at[b], shared.at[idxr.at[b]], ssem, add=True)
        return 0
    lax.fori_loop(0, n_chunks // NBUF, outer, 0)
    for k in range(NBUF - GAHEAD):
        j = n_chunks - (NBUF - GAHEAD) + k
        scatter_desc(j, j % NBUF).wait()
    plsc.subcore_barrier()

    for k in range(ROWS_PER_TILE // ZROWS):
        r0 = s * ROWS_PER_TILE + k * ZROWS
        pltpu.sync_copy(shared.at[pl.ds(r0, ZROWS), :],
                        out_hbm.at[pl.ds(r0, ZROWS), pl.ds(c * DH, DH)])


def kernel(edge_index, edge_weight, X):
    rows = edge_index[0].astype(jnp.int32)
    cols = edge_index[1].astype(jnp.int32)
    w = edge_weight.astype(jnp.float32)
    n_edges = rows.shape[0]

    epg = NS * CHUNK * NBUF
    n_pad = (-n_edges) % epg
    if n_pad:
        rows = jnp.concatenate([rows, jnp.zeros((n_pad,), jnp.int32)])
        cols = jnp.concatenate([cols, jnp.zeros((n_pad,), jnp.int32)])
        w = jnp.concatenate([w, jnp.zeros((n_pad,), jnp.float32)])
    e_pad = n_edges + n_pad
    n_chunks = e_pad // (NS * CHUNK)
    per_tile = n_chunks * CHUNK

    pk = jnp.left_shift(rows, PBITS) | cols
    pk2 = pk.reshape(NS, per_tile)
    w2 = w.reshape(NS, per_tile)
    x2 = jnp.concatenate([X[:, :DH], X[:, DH:]], axis=0)

    mesh = plsc.VectorSubcoreMesh(core_axis_name="c", subcore_axis_name="s")
    f = pl.kernel(
        functools.partial(_sc_body, n_chunks),
        out_type=jax.ShapeDtypeStruct((N_NODES, D), jnp.float32),
        mesh=mesh,
        compiler_params=pltpu.CompilerParams(use_tc_tiling_on_sc=False,
                                             needs_layout_passes=False),
        scratch_types=[
            pltpu.VMEM((per_tile,), jnp.int32),
            pltpu.VMEM((per_tile,), jnp.float32),
            pltpu.VMEM((NBUF, CHUNK), jnp.int32),
            pltpu.VMEM((NBUF, CHUNK), jnp.int32),
            pltpu.VMEM((NBUF, CHUNK, DH), jnp.float32),
            pltpu.VMEM_SHARED((N_NODES, DH), jnp.float32),
            pltpu.SemaphoreType.DMA,
            pltpu.SemaphoreType.DMA,
        ],
    )
    return f(pk2, w2, x2)


# speedup vs baseline: 1.0042x; 1.0042x over previous
import functools

import jax
import jax.numpy as jnp
from jax import lax
from jax.experimental import pallas as pl
from jax.experimental.pallas import tpu as pltpu
from jax.experimental.pallas import tpu_sc as plsc

N_NODES = 10000
D = 128
DH = D // 2
NC = 2
NS = 16
L = 16
CHUNK = 128
NBUF = 5
GAHEAD = 2
PBITS = 14
PMASK = (1 << PBITS) - 1

ROWS_PER_TILE = N_NODES // NS
ZROWS = 125


def _sc_body(n_chunks, pk_hbm, w_hbm, x_hbm, out_hbm,
             pk, wv, idxc, idxr, gb, shared, gsem, ssem):
    c = lax.axis_index("c")
    s = lax.axis_index("s")

    pltpu.sync_copy(pk_hbm.at[s], pk)
    pltpu.sync_copy(w_hbm.at[s], wv)

    zeros = jnp.zeros((L,), jnp.float32)

    def zrow(i, _):
        for v in range(DH // L):
            gb[0, i, pl.ds(v * L, L)] = zeros
        return 0
    lax.fori_loop(0, CHUNK, zrow, 0)
    for k in range(ROWS_PER_TILE // ZROWS):
        pltpu.sync_copy(gb.at[0, pl.ds(0, ZROWS), :],
                        shared.at[pl.ds(s * ROWS_PER_TILE + k * ZROWS, ZROWS), :])
    plsc.subcore_barrier()

    col_off = c * N_NODES

    def unpack(k, slot):
        for v in range(CHUNK // L):
            p = pk[pl.ds(k * CHUNK + v * L, L)]
            idxr[slot, pl.ds(v * L, L)] = lax.shift_right_logical(p, PBITS)
            idxc[slot, pl.ds(v * L, L)] = (p & PMASK) + col_off

    def gather_desc(j, b):
        return pltpu.make_async_copy(x_hbm.at[idxc.at[b]], gb.at[b], gsem)

    def scatter_desc(j, b):
        return pltpu.make_async_copy(gb.at[b], shared.at[idxr.at[b]], ssem)

    for b in range(GAHEAD):
        unpack(b, b)
        gather_desc(b, b).start()

    def outer(jo, _):
        for b in range(NBUF):
            j = jo * NBUF + b
            gather_desc(j, b).wait()
            jbase = j * CHUNK

            def escale(e, _):
                w16 = plsc.load_gather(wv, [jnp.broadcast_to(jbase + e, (L,))])
                for v in range(DH // L):
                    gb[b, e, pl.ds(v * L, L)] = gb[b, e, pl.ds(v * L, L)] * w16
                return 0
            lax.fori_loop(0, CHUNK, escale, 0, unroll=2)

            @pl.when(j >= NBUF - GAHEAD)
            def _():
                scatter_desc(j - (NBUF - GAHEAD), (b + GAHEAD) % NBUF).wait()

            @pl.when(j + GAHEAD < n_chunks)
            def _():
                unpack(j + GAHEAD, (b + GAHEAD) % NBUF)
                gather_desc(j + GAHEAD, (b + GAHEAD) % NBUF).start()
            pltpu.async_copy(gb.at[b], shared.at[idxr.at[b]], ssem, add=True)
        return 0
    lax.fori_loop(0, n_chunks // NBUF, outer, 0)
    for k in range(NBUF - GAHEAD):
        j = n_chunks - (NBUF - GAHEAD) + k
        scatter_desc(j, j % NBUF).wait()
    plsc.subcore_barrier()

    for k in range(ROWS_PER_TILE // ZROWS):
        r0 = s * ROWS_PER_TILE + k * ZROWS
        pltpu.sync_copy(shared.at[pl.ds(r0, ZROWS), :],
                        out_hbm.at[pl.ds(r0, ZROWS), pl.ds(c * DH, DH)])


def kernel(edge_index, edge_weight, X):
    rows = edge_index[0].astype(jnp.int32)
    cols = edge_index[1].astype(jnp.int32)
    w = edge_weight.astype(jnp.float32)
    n_edges = rows.shape[0]

    epg = NS * CHUNK * NBUF
    n_pad = (-n_edges) % epg
    if n_pad:
        rows = jnp.concatenate([rows, jnp.zeros((n_pad,), jnp.int32)])
        cols = jnp.concatenate([cols, jnp.zeros((n_pad,), jnp.int32)])
        w = jnp.concatenate([w, jnp.zeros((n_pad,), jnp.float32)])
    e_pad = n_edges + n_pad
    n_chunks = e_pad // (NS * CHUNK)
    per_tile = n_chunks * CHUNK

    pk = jnp.left_shift(rows, PBITS) | cols
    pk2 = pk.reshape(NS, per_tile)
    w2 = w.reshape(NS, per_tile)
    x2 = jnp.concatenate([X[:, :DH], X[:, DH:]], axis=0)

    mesh = plsc.VectorSubcoreMesh(core_axis_name="c", subcore_axis_name="s")
    f = pl.kernel(
        functools.partial(_sc_body, n_chunks),
        out_type=jax.ShapeDtypeStruct((N_NODES, D), jnp.float32),
        mesh=mesh,
        compiler_params=pltpu.CompilerParams(use_tc_tiling_on_sc=False,
                                             needs_layout_passes=False),
        scratch_types=[
            pltpu.VMEM((per_tile,), jnp.int32),
            pltpu.VMEM((per_tile,), jnp.float32),
            pltpu.VMEM((NBUF, CHUNK), jnp.int32),
            pltpu.VMEM((NBUF, CHUNK), jnp.int32),
            pltpu.VMEM((NBUF, CHUNK, DH), jnp.float32),
            pltpu.VMEM_SHARED((N_NODES, DH), jnp.float32),
            pltpu.SemaphoreType.DMA,
            pltpu.SemaphoreType.DMA,
        ],
    )
    return f(pk2, w2, x2)


# speedup vs baseline: 1.7379x; 1.7307x over previous
import functools

import jax
import jax.numpy as jnp
from jax import lax
from jax.experimental import pallas as pl
from jax.experimental.pallas import tpu as pltpu
from jax.experimental.pallas import tpu_sc as plsc

N_NODES = 10000
D = 128
DH = D // 2
NC = 2
NS = 16
L = 16
CHUNK = 128
SB = 4
SRING = 3

ROWS_PER_TILE = N_NODES // NS
ZROWS = 125


def _sc_body(n_sb, rows_hbm, cols_hbm, w_hbm, x_hbm, out_hbm,
             rring, cring, wring, gb, xs, acc, gsem, ssem, isem):
    c = lax.axis_index("c")
    s = lax.axis_index("s")

    r0 = s * ROWS_PER_TILE
    pltpu.sync_copy(x_hbm.at[c, pl.ds(r0, ROWS_PER_TILE), :],
                    xs.at[pl.ds(r0, ROWS_PER_TILE), :])

    zeros = jnp.zeros((L,), jnp.float32)

    def zrow(i, _):
        for v in range(DH // L):
            gb[0, i, pl.ds(v * L, L)] = zeros
        return 0
    lax.fori_loop(0, CHUNK, zrow, 0)
    for k in range(ROWS_PER_TILE // ZROWS):
        pltpu.sync_copy(gb.at[0, pl.ds(0, ZROWS), :],
                        acc.at[pl.ds(r0 + k * ZROWS, ZROWS), :])
    plsc.subcore_barrier()

    def stage_descs(k, sl):
        return (pltpu.make_async_copy(rows_hbm.at[s, k], rring.at[sl], isem),
                pltpu.make_async_copy(cols_hbm.at[s, k], cring.at[sl], isem),
                pltpu.make_async_copy(w_hbm.at[s, k], wring.at[sl], isem))

    def gather_desc(sl, cb, b):
        return pltpu.make_async_copy(xs.at[cring.at[sl, cb]], gb.at[b], gsem)

    def scatter_desc(sl, cb, b):
        return pltpu.make_async_copy(gb.at[b], acc.at[rring.at[sl, cb]], ssem)

    for d in stage_descs(0, 0):
        d.start()
    for d in stage_descs(1, 1):
        d.start()
    for d in stage_descs(0, 0) + stage_descs(1, 1):
        d.wait()
    for d in stage_descs(2, 2):
        d.start()
    gather_desc(0, 0, 0).start()
    gather_desc(0, 1, 1).start()

    def outer(k, _):
        sl0 = lax.rem(k, SRING)
        sl1 = lax.rem(k + 1, SRING)
        sl2 = lax.rem(k + 2, SRING)

        @pl.when(jnp.logical_and(k >= 1, k + 1 < n_sb))
        def _():
            for d in stage_descs(k + 1, sl1):
                d.wait()

        for cb in range(SB):
            j = k * SB + cb
            b = cb
            gather_desc(sl0, cb, b).wait()
            ebase = cb * CHUNK

            def escale(e, _):
                w16 = plsc.load_gather(
                    wring, [jnp.broadcast_to(sl0, (L,)),
                            jnp.broadcast_to(ebase + e, (L,))])
                for v in range(DH // L):
                    gb[b, e, pl.ds(v * L, L)] = gb[b, e, pl.ds(v * L, L)] * w16
                return 0
            lax.fori_loop(0, CHUNK, escale, 0, unroll=2)

            nb = (cb + 2) % SB
            pcb = cb - 2 if cb >= 2 else cb + 2
            psl = sl0 if cb >= 2 else lax.rem(k + SRING - 1, SRING)

            @pl.when(j >= 2)
            def _():
                scatter_desc(psl, pcb, nb).wait()

            if cb == 2:
                @pl.when(jnp.logical_and(k >= 1, k + 2 < n_sb))
                def _():
                    for d in stage_descs(k + 2, sl2):
                        d.start()

            ncb = cb + 2 if cb < 2 else cb - 2
            nsl = sl0 if cb < 2 else sl1

            @pl.when(j + 2 < n_sb * SB)
            def _():
                gather_desc(nsl, ncb, nb).start()

            pltpu.async_copy(gb.at[b], acc.at[rring.at[sl0, cb]], ssem,
                             add=True)
        return 0
    lax.fori_loop(0, n_sb, outer, 0)
    last_sl = lax.rem(n_sb - 1, SRING)
    scatter_desc(last_sl, SB - 2, SB - 2).wait()
    scatter_desc(last_sl, SB - 1, SB - 1).wait()
    plsc.subcore_barrier()

    for k in range(ROWS_PER_TILE // ZROWS):
        rr = r0 + k * ZROWS
        pltpu.sync_copy(acc.at[pl.ds(rr, ZROWS), :],
                        out_hbm.at[pl.ds(rr, ZROWS), pl.ds(c * DH, DH)])


def kernel(edge_index, edge_weight, X):
    rows = edge_index[0].astype(jnp.int32)
    cols = edge_index[1].astype(jnp.int32)
    w = edge_weight.astype(jnp.float32)
    n_edges = rows.shape[0]

    epg = NS * CHUNK * SB
    n_pad = (-n_edges) % epg
    if n_pad:
        rows = jnp.concatenate([rows, jnp.zeros((n_pad,), jnp.int32)])
        cols = jnp.concatenate([cols, jnp.zeros((n_pad,), jnp.int32)])
        w = jnp.concatenate([w, jnp.zeros((n_pad,), jnp.float32)])
    e_pad = n_edges + n_pad
    n_sb = e_pad // epg
    rows4 = rows.reshape(NS, n_sb, SB, CHUNK)
    cols4 = cols.reshape(NS, n_sb, SB, CHUNK)
    w3 = w.reshape(NS, n_sb, SB * CHUNK)
    x3 = jnp.stack([X[:, :DH], X[:, DH:]])

    mesh = plsc.VectorSubcoreMesh(core_axis_name="c", subcore_axis_name="s")
    f = pl.kernel(
        functools.partial(_sc_body, n_sb),
        out_type=jax.ShapeDtypeStruct((N_NODES, D), jnp.float32),
        mesh=mesh,
        compiler_params=pltpu.CompilerParams(use_tc_tiling_on_sc=False,
                                             needs_layout_passes=False),
        scratch_types=[
            pltpu.VMEM((SRING, SB, CHUNK), jnp.int32),
            pltpu.VMEM((SRING, SB, CHUNK), jnp.int32),
            pltpu.VMEM((SRING, SB * CHUNK), jnp.float32),
            pltpu.VMEM((SB, CHUNK, DH), jnp.float32),
            pltpu.VMEM_SHARED((N_NODES, DH), jnp.float32),
            pltpu.VMEM_SHARED((N_NODES, DH), jnp.float32),
            pltpu.SemaphoreType.DMA,
            pltpu.SemaphoreType.DMA,
            pltpu.SemaphoreType.DMA,
        ],
    )
    return f(rows4, cols4, w3, x3)
